# baseline (speedup 1.0000x reference)
import functools

import jax
import jax.numpy as jnp
from jax import lax
from jax.experimental import pallas as pl
from jax.experimental.pallas import tpu as pltpu
from jax.experimental.pallas import tpu_sc as plsc

N = 10000
E = 320000
H = 128
L = 3
G = 64
C = 10

NPAD = 10240
NC = 2
NS = 16
NW = NC * NS
CHUNK = 128
EPW = 10112
NCHUNK = EPW // CHUNK
EPAD = NW * EPW
ROWS_PER_TILE = NPAD // NS
BLK = 2048

_HI = jax.lax.Precision.HIGHEST


def _dot(a, b):
    return jax.lax.dot_general(a, b, (((1,), (0,)), ((), ())),
                               precision=_HI, preferred_element_type=jnp.float32)


def _edge_body(m_hbm, src_hbm, dst_hbm, ewx_hbm, zrows_hbm, out_hbm,
               src_v, dst0, ewx0, rows0, agg_sh,
               sg0, sg1, se0, sd0, ss0):
    c = lax.axis_index("c")
    s = lax.axis_index("s")
    wid = s * NC + c

    pltpu.sync_copy(src_hbm.at[wid], src_v)
    pltpu.sync_copy(zrows_hbm, agg_sh.at[pl.ds(s * ROWS_PER_TILE, ROWS_PER_TILE)])
    plsc.subcore_barrier()

    bufs = ((rows0, ewx0, dst0, sg0, se0, sd0, ss0),)

    def issue_in(j, b):
        rows_b, ewx_b, dst_b, sg, se, sd, ss = bufs[b]
        pltpu.async_copy(m_hbm.at[src_v.at[j]], rows_b, sg)
        pltpu.async_copy(ewx_hbm.at[wid, j], ewx_b, se)
        pltpu.async_copy(dst_hbm.at[wid, j], dst_b, sd)

    def wait_in(j, b):
        rows_b, ewx_b, dst_b, sg, se, sd, ss = bufs[b]
        pltpu.make_async_copy(m_hbm.at[src_v.at[j]], rows_b, sg).wait()
        pltpu.make_async_copy(ewx_hbm.at[wid, j], ewx_b, se).wait()

    def multiply(b):
        rows_b, ewx_b = bufs[b][0], bufs[b][1]

        def g_step(g, carry):
            base = g * 8
            for k in range(8):
                w = ewx_b[g, pl.ds(k * 16, 16)]
                for cb in range(H // 16):
                    sl = pl.ds(cb * 16, 16)
                    rows_b[base + k, sl] = rows_b[base + k, sl] * w
            return carry

        lax.fori_loop(0, CHUNK // 8, g_step, 0, unroll=False)

    def issue_scatter(j, b):
        rows_b, ewx_b, dst_b, sg, se, sd, ss = bufs[b]
        pltpu.make_async_copy(dst_hbm.at[wid, j], dst_b, sd).wait()
        pltpu.async_copy(rows_b, agg_sh.at[dst_b.at[0]], ss, add=True)

    def wait_scatter(b):
        rows_b, ewx_b, dst_b, sg, se, sd, ss = bufs[b]
        pltpu.make_async_copy(rows_b, agg_sh.at[dst_b.at[0]], ss).wait()

    def chunk_step(j, carry):
        ga = pltpu.async_copy(m_hbm.at[src_v.at[j, pl.ds(0, CHUNK // 2)]],
                              rows0.at[pl.ds(0, CHUNK // 2)], sg0)
        gb = pltpu.async_copy(m_hbm.at[src_v.at[j, pl.ds(CHUNK // 2, CHUNK // 2)]],
                              rows0.at[pl.ds(CHUNK // 2, CHUNK // 2)], sg1)
        e_ = pltpu.async_copy(ewx_hbm.at[wid, j], ewx0, se0)
        d = pltpu.async_copy(dst_hbm.at[wid, j], dst0, sd0)
        ga.wait()
        gb.wait()
        e_.wait()
        multiply(0)
        d.wait()
        sc = pltpu.async_copy(rows0, agg_sh.at[dst0.at[0]], ss0, add=True)
        sc.wait()
        return carry

    lax.fori_loop(0, NCHUNK, chunk_step, 0, unroll=False)
    plsc.subcore_barrier()
    sl = pl.ds(s * ROWS_PER_TILE, ROWS_PER_TILE)
    pltpu.sync_copy(agg_sh.at[sl], out_hbm.at[c, sl])


@functools.partial(jax.jit, static_argnames=())
def _edge_call(m, src3, dst3, ew3, zrows):
    mesh = plsc.VectorSubcoreMesh(core_axis_name="c", subcore_axis_name="s")
    k = pl.kernel(
        _edge_body,
        out_type=jax.ShapeDtypeStruct((NC, NPAD, H), jnp.float32),
        mesh=mesh,
        scratch_types=[
            pltpu.VMEM((NCHUNK, CHUNK), jnp.int32),
            pltpu.VMEM((1, CHUNK), jnp.int32),
            pltpu.VMEM((16, CHUNK), jnp.float32),
            pltpu.VMEM((CHUNK, H), jnp.float32),
            pltpu.VMEM_SHARED((NPAD, H), jnp.float32),
        ] + [pltpu.SemaphoreType.DMA] * 5,
    )
    return k(m, src3, dst3, ew3, zrows)


def _mm_body(h_ref, w_ref, o_ref):
    o_ref[...] = _dot(h_ref[...], w_ref[...])


def _mm(h, w, blk=BLK):
    return pl.pallas_call(
        _mm_body,
        grid=(NPAD // blk,),
        in_specs=[
            pl.BlockSpec((blk, H), lambda i: (i, 0)),
            pl.BlockSpec((H, H), lambda i: (0, 0)),
        ],
        out_specs=pl.BlockSpec((blk, H), lambda i: (i, 0)),
        out_shape=jax.ShapeDtypeStruct((NPAD, H), jnp.float32),
    )(h, w)


def _gru_body(parts_ref, h_ref, wih_t_ref, whh_t_ref, bih_ref, bhh_ref,
              wnext_ref, hout_ref, mout_ref):
    agg = parts_ref[0] + parts_ref[1]
    h = h_ref[...]
    gi = _dot(agg, wih_t_ref[...]) + bih_ref[...]
    gh = _dot(h, whh_t_ref[...]) + bhh_ref[...]
    i_r, i_z, i_n = gi[:, :H], gi[:, H:2 * H], gi[:, 2 * H:]
    h_r, h_z, h_n = gh[:, :H], gh[:, H:2 * H], gh[:, 2 * H:]
    r = jax.nn.sigmoid(i_r + h_r)
    z = jax.nn.sigmoid(i_z + h_z)
    n = jnp.tanh(i_n + r * h_n)
    hn = (1.0 - z) * n + z * h
    hout_ref[...] = hn
    mout_ref[...] = _dot(hn, wnext_ref[...])


def _gru(parts, h, wih_t, whh_t, bih, bhh, wnext, blk=BLK):
    return pl.pallas_call(
        _gru_body,
        grid=(NPAD // blk,),
        in_specs=[
            pl.BlockSpec((NC, blk, H), lambda i: (0, i, 0)),
            pl.BlockSpec((blk, H), lambda i: (i, 0)),
            pl.BlockSpec((H, 3 * H), lambda i: (0, 0)),
            pl.BlockSpec((H, 3 * H), lambda i: (0, 0)),
            pl.BlockSpec((1, 3 * H), lambda i: (0, 0)),
            pl.BlockSpec((1, 3 * H), lambda i: (0, 0)),
            pl.BlockSpec((H, H), lambda i: (0, 0)),
        ],
        out_specs=[
            pl.BlockSpec((blk, H), lambda i: (i, 0)),
            pl.BlockSpec((blk, H), lambda i: (i, 0)),
        ],
        out_shape=[
            jax.ShapeDtypeStruct((NPAD, H), jnp.float32),
            jax.ShapeDtypeStruct((NPAD, H), jnp.float32),
        ],
    )(parts, h, wih_t, whh_t, bih, bhh, wnext)


def _pool_body(h_ref, batch_ref, win_t_ref, bin_ref, wmid_t_ref, bmid_ref,
               wout_tp_ref, bout_p_ref, out_ref, pooled_acc, cnt_acc):
    i = pl.program_id(0)
    nb = pl.num_programs(0)

    @pl.when(i == 0)
    def _init():
        pooled_acc[...] = jnp.zeros_like(pooled_acc)
        cnt_acc[...] = jnp.zeros_like(cnt_acc)

    b = batch_ref[0]
    giota = jax.lax.broadcasted_iota(jnp.int32, (G, b.shape[1]), 0)
    maskf = (giota == b).astype(jnp.float32)
    pooled_acc[...] += _dot(maskf, h_ref[...])
    cnt = jnp.sum(maskf, axis=1, keepdims=True)
    cnt_acc[...] += jnp.broadcast_to(cnt, cnt_acc.shape)

    @pl.when(i == nb - 1)
    def _final():
        pooled = pooled_acc[...] / jnp.maximum(cnt_acc[...], 1.0)
        h1 = jnp.maximum(_dot(pooled, win_t_ref[...]) + bin_ref[...], 0.0)
        h2 = jnp.maximum(_dot(h1, wmid_t_ref[...]) + bmid_ref[...], 0.0)
        out_ref[...] = _dot(h2, wout_tp_ref[...]) + bout_p_ref[...]


def _pool_head(h, batch3, win_t, bin_, wmid_t, bmid, wout_tp, bout_p, blk=BLK):
    nb = NPAD // blk
    return pl.pallas_call(
        _pool_body,
        grid=(nb,),
        in_specs=[
            pl.BlockSpec((blk, H), lambda i: (i, 0)),
            pl.BlockSpec((1, 1, blk), lambda i: (i, 0, 0)),
            pl.BlockSpec((H, H), lambda i: (0, 0)),
            pl.BlockSpec((1, H), lambda i: (0, 0)),
            pl.BlockSpec((H, H), lambda i: (0, 0)),
            pl.BlockSpec((1, H), lambda i: (0, 0)),
            pl.BlockSpec((H, H), lambda i: (0, 0)),
            pl.BlockSpec((1, H), lambda i: (0, 0)),
        ],
        out_specs=pl.BlockSpec((G, H), lambda i: (0, 0)),
        out_shape=jax.ShapeDtypeStruct((G, H), jnp.float32),
        scratch_shapes=[
            pltpu.VMEM((G, H), jnp.float32),
            pltpu.VMEM((G, H), jnp.float32),
        ],
    )(h, batch3, win_t, bin_, wmid_t, bmid, wout_tp, bout_p)


def kernel(inputs, edge_index, batch, edge_weight, conv_weight, w_ih, w_hh,
           b_ih, b_hh, W_in, b_in, W_mid, b_mid, W_out, b_out):
    f32 = jnp.float32
    h = jnp.zeros((NPAD, H), f32).at[:N].set(inputs)
    src = edge_index[0]
    dst = edge_index[1]
    ew = edge_weight
    pad_e = EPAD - E
    src3 = jnp.concatenate([src, jnp.zeros((pad_e,), jnp.int32)]).reshape(NW, NCHUNK, CHUNK)
    dst3 = jnp.concatenate([dst, jnp.zeros((pad_e,), jnp.int32)]).reshape(NW, NCHUNK, 1, CHUNK)
    ew_p = jnp.concatenate([ew, jnp.zeros((pad_e,), f32)])
    ew3 = jnp.broadcast_to(ew_p[:, None], (EPAD, 16)).reshape(NW, NCHUNK, 16, CHUNK)
    zrows = jnp.zeros((ROWS_PER_TILE, H), f32)
    batch3 = jnp.full((NPAD,), jnp.int32(2**30), jnp.int32).at[:N].set(batch)
    batch3 = batch3.reshape(NPAD // BLK, 1, BLK)
    wih_t = w_ih.T
    whh_t = w_hh.T
    bih = b_ih.reshape(1, 3 * H)
    bhh = b_hh.reshape(1, 3 * H)
    win_t = W_in.T
    bin_ = b_in.reshape(1, H)
    wmid_t = W_mid.T
    bmid = b_mid.reshape(1, H)
    wout_tp = jnp.zeros((H, H), f32).at[:, :C].set(W_out.T)
    bout_p = jnp.zeros((1, H), f32).at[0, :C].set(b_out)

    m = _mm(h, conv_weight[0])
    for i in range(L):
        parts = _edge_call(m, src3, dst3, ew3, zrows)
        wnext = conv_weight[(i + 1) % L]
        h, m = _gru(parts, h, wih_t, whh_t, bih, bhh, wnext)

    out_p = _pool_head(h, batch3, win_t, bin_, wmid_t, bmid, wout_tp, bout_p)
    return out_p[:, :C]


# speedup vs baseline: 1.2706x; 1.2706x over previous
import functools

import jax
import jax.numpy as jnp
from jax import lax
from jax.experimental import pallas as pl
from jax.experimental.pallas import tpu as pltpu
from jax.experimental.pallas import tpu_sc as plsc

N = 10000
E = 320000
H = 128
L = 3
G = 64
C = 10

NPAD = 10240
NC = 2
NS = 16
NW = NC * NS
CHUNK = 128
EPW = 10112
NCHUNK = EPW // CHUNK
EPAD = NW * EPW
ROWS_PER_TILE = NPAD // NS
BLK = 2048

_HI = jax.lax.Precision.HIGHEST


def _dot(a, b):
    return jax.lax.dot_general(a, b, (((1,), (0,)), ((), ())),
                               precision=_HI, preferred_element_type=jnp.float32)


def _edge_body(m_hbm, src_hbm, dst_hbm, ewx_hbm, zrows_hbm, out_hbm,
               src_v, dst2, ewx0, ewx1, rows0, rows1, agg_sh,
               sg0, sg1, se0, se1, sd0, sd1, ss0, ss1):
    c = lax.axis_index("c")
    s = lax.axis_index("s")
    wid = s * NC + c

    pltpu.sync_copy(src_hbm.at[wid], src_v)
    pltpu.sync_copy(zrows_hbm, agg_sh.at[pl.ds(s * ROWS_PER_TILE, ROWS_PER_TILE)])
    plsc.subcore_barrier()

    bufs = ((rows0, ewx0, dst2.at[0], sg0, se0, sd0, ss0),
            (rows1, ewx1, dst2.at[1], sg1, se1, sd1, ss1))

    def issue_in(j, b):
        rows_b, ewx_b, dst_b, sg, se, sd, ss = bufs[b]
        pltpu.async_copy(m_hbm.at[src_v.at[j]], rows_b, sg)
        pltpu.async_copy(ewx_hbm.at[wid, j], ewx_b, se)
        pltpu.async_copy(dst_hbm.at[wid, j, 0], dst_b, sd)

    def wait_in(j, b):
        rows_b, ewx_b, dst_b, sg, se, sd, ss = bufs[b]
        pltpu.make_async_copy(m_hbm.at[src_v.at[j]], rows_b, sg).wait()
        pltpu.make_async_copy(ewx_hbm.at[wid, j], ewx_b, se).wait()

    def multiply(b):
        rows_b, ewx_b = bufs[b][0], bufs[b][1]

        def g_step(g, carry):
            base = g * 8
            for k in range(8):
                w = ewx_b[g, pl.ds(k * 16, 16)]
                for cb in range(H // 16):
                    sl = pl.ds(cb * 16, 16)
                    rows_b[base + k, sl] = rows_b[base + k, sl] * w
            return carry

        lax.fori_loop(0, CHUNK // 8, g_step, 0, unroll=False)

    def issue_scatter(j, b):
        rows_b, ewx_b, dst_b, sg, se, sd, ss = bufs[b]
        pltpu.make_async_copy(dst_hbm.at[wid, j, 0], dst_b, sd).wait()
        pltpu.async_copy(rows_b, agg_sh.at[dst_b], ss, add=True)

    def wait_scatter(b):
        rows_b, ewx_b, dst_b, sg, se, sd, ss = bufs[b]
        pltpu.make_async_copy(rows_b, agg_sh.at[dst_b], ss).wait()

    npair = NCHUNK // 2
    issue_in(0, 0)

    def pair(jj, carry):
        j0 = 2 * jj
        j1 = j0 + 1
        wait_in(j0, 0)

        @pl.when(jj > 0)
        def _():
            wait_scatter(1)

        issue_in(j1, 1)
        multiply(0)
        issue_scatter(j0, 0)
        wait_in(j1, 1)
        wait_scatter(0)
        issue_in(j0 + 2, 0)
        multiply(1)
        issue_scatter(j1, 1)
        return carry

    lax.fori_loop(0, npair, pair, 0, unroll=False)
    wait_in(NCHUNK - 1, 0)
    wait_scatter(1)
    multiply(0)
    issue_scatter(NCHUNK - 1, 0)
    wait_scatter(0)
    plsc.subcore_barrier()
    sl = pl.ds(s * ROWS_PER_TILE, ROWS_PER_TILE)
    pltpu.sync_copy(agg_sh.at[sl], out_hbm.at[c, sl])


@functools.partial(jax.jit, static_argnames=())
def _edge_call(m, src3, dst3, ew3, zrows):
    mesh = plsc.VectorSubcoreMesh(core_axis_name="c", subcore_axis_name="s")
    k = pl.kernel(
        _edge_body,
        out_type=jax.ShapeDtypeStruct((NC, NPAD, H), jnp.float32),
        mesh=mesh,
        scratch_types=[
            pltpu.VMEM((NCHUNK, CHUNK), jnp.int32),
            pltpu.VMEM((2, CHUNK), jnp.int32),
            pltpu.VMEM((16, CHUNK), jnp.float32),
            pltpu.VMEM((16, CHUNK), jnp.float32),
            pltpu.VMEM((CHUNK, H), jnp.float32),
            pltpu.VMEM((CHUNK, H), jnp.float32),
            pltpu.VMEM_SHARED((NPAD, H), jnp.float32),
        ] + [pltpu.SemaphoreType.DMA] * 8,
    )
    return k(m, src3, dst3, ew3, zrows)


def _mm_body(h_ref, w_ref, o_ref):
    o_ref[...] = _dot(h_ref[...], w_ref[...])


def _mm(h, w, blk=BLK):
    return pl.pallas_call(
        _mm_body,
        grid=(NPAD // blk,),
        in_specs=[
            pl.BlockSpec((blk, H), lambda i: (i, 0)),
            pl.BlockSpec((H, H), lambda i: (0, 0)),
        ],
        out_specs=pl.BlockSpec((blk, H), lambda i: (i, 0)),
        out_shape=jax.ShapeDtypeStruct((NPAD, H), jnp.float32),
    )(h, w)


def _gru_body(parts_ref, h_ref, wih_t_ref, whh_t_ref, bih_ref, bhh_ref,
              wnext_ref, hout_ref, mout_ref):
    agg = parts_ref[0] + parts_ref[1]
    h = h_ref[...]
    gi = _dot(agg, wih_t_ref[...]) + bih_ref[...]
    gh = _dot(h, whh_t_ref[...]) + bhh_ref[...]
    i_r, i_z, i_n = gi[:, :H], gi[:, H:2 * H], gi[:, 2 * H:]
    h_r, h_z, h_n = gh[:, :H], gh[:, H:2 * H], gh[:, 2 * H:]
    r = jax.nn.sigmoid(i_r + h_r)
    z = jax.nn.sigmoid(i_z + h_z)
    n = jnp.tanh(i_n + r * h_n)
    hn = (1.0 - z) * n + z * h
    hout_ref[...] = hn
    mout_ref[...] = _dot(hn, wnext_ref[...])


def _gru(parts, h, wih_t, whh_t, bih, bhh, wnext, blk=BLK):
    return pl.pallas_call(
        _gru_body,
        grid=(NPAD // blk,),
        in_specs=[
            pl.BlockSpec((NC, blk, H), lambda i: (0, i, 0)),
            pl.BlockSpec((blk, H), lambda i: (i, 0)),
            pl.BlockSpec((H, 3 * H), lambda i: (0, 0)),
            pl.BlockSpec((H, 3 * H), lambda i: (0, 0)),
            pl.BlockSpec((1, 3 * H), lambda i: (0, 0)),
            pl.BlockSpec((1, 3 * H), lambda i: (0, 0)),
            pl.BlockSpec((H, H), lambda i: (0, 0)),
        ],
        out_specs=[
            pl.BlockSpec((blk, H), lambda i: (i, 0)),
            pl.BlockSpec((blk, H), lambda i: (i, 0)),
        ],
        out_shape=[
            jax.ShapeDtypeStruct((NPAD, H), jnp.float32),
            jax.ShapeDtypeStruct((NPAD, H), jnp.float32),
        ],
    )(parts, h, wih_t, whh_t, bih, bhh, wnext)


def _pool_body(h_ref, batch_ref, win_t_ref, bin_ref, wmid_t_ref, bmid_ref,
               wout_tp_ref, bout_p_ref, out_ref, pooled_acc, cnt_acc):
    i = pl.program_id(0)
    nb = pl.num_programs(0)

    @pl.when(i == 0)
    def _init():
        pooled_acc[...] = jnp.zeros_like(pooled_acc)
        cnt_acc[...] = jnp.zeros_like(cnt_acc)

    b = batch_ref[0]
    giota = jax.lax.broadcasted_iota(jnp.int32, (G, b.shape[1]), 0)
    maskf = (giota == b).astype(jnp.float32)
    pooled_acc[...] += _dot(maskf, h_ref[...])
    cnt = jnp.sum(maskf, axis=1, keepdims=True)
    cnt_acc[...] += jnp.broadcast_to(cnt, cnt_acc.shape)

    @pl.when(i == nb - 1)
    def _final():
        pooled = pooled_acc[...] / jnp.maximum(cnt_acc[...], 1.0)
        h1 = jnp.maximum(_dot(pooled, win_t_ref[...]) + bin_ref[...], 0.0)
        h2 = jnp.maximum(_dot(h1, wmid_t_ref[...]) + bmid_ref[...], 0.0)
        out_ref[...] = _dot(h2, wout_tp_ref[...]) + bout_p_ref[...]


def _pool_head(h, batch3, win_t, bin_, wmid_t, bmid, wout_tp, bout_p, blk=BLK):
    nb = NPAD // blk
    return pl.pallas_call(
        _pool_body,
        grid=(nb,),
        in_specs=[
            pl.BlockSpec((blk, H), lambda i: (i, 0)),
            pl.BlockSpec((1, 1, blk), lambda i: (i, 0, 0)),
            pl.BlockSpec((H, H), lambda i: (0, 0)),
            pl.BlockSpec((1, H), lambda i: (0, 0)),
            pl.BlockSpec((H, H), lambda i: (0, 0)),
            pl.BlockSpec((1, H), lambda i: (0, 0)),
            pl.BlockSpec((H, H), lambda i: (0, 0)),
            pl.BlockSpec((1, H), lambda i: (0, 0)),
        ],
        out_specs=pl.BlockSpec((G, H), lambda i: (0, 0)),
        out_shape=jax.ShapeDtypeStruct((G, H), jnp.float32),
        scratch_shapes=[
            pltpu.VMEM((G, H), jnp.float32),
            pltpu.VMEM((G, H), jnp.float32),
        ],
    )(h, batch3, win_t, bin_, wmid_t, bmid, wout_tp, bout_p)


def kernel(inputs, edge_index, batch, edge_weight, conv_weight, w_ih, w_hh,
           b_ih, b_hh, W_in, b_in, W_mid, b_mid, W_out, b_out):
    f32 = jnp.float32
    h = jnp.zeros((NPAD, H), f32).at[:N].set(inputs)
    src = edge_index[0]
    dst = edge_index[1]
    ew = edge_weight
    pad_e = EPAD - E
    src3 = jnp.concatenate([src, jnp.zeros((pad_e,), jnp.int32)]).reshape(NW, NCHUNK, CHUNK)
    dst3 = jnp.concatenate([dst, jnp.zeros((pad_e,), jnp.int32)]).reshape(NW, NCHUNK, 1, CHUNK)
    ew_p = jnp.concatenate([ew, jnp.zeros((pad_e,), f32)])
    ew3 = jnp.broadcast_to(ew_p[:, None], (EPAD, 16)).reshape(NW, NCHUNK, 16, CHUNK)
    zrows = jnp.zeros((ROWS_PER_TILE, H), f32)
    batch3 = jnp.full((NPAD,), jnp.int32(2**30), jnp.int32).at[:N].set(batch)
    batch3 = batch3.reshape(NPAD // BLK, 1, BLK)
    wih_t = w_ih.T
    whh_t = w_hh.T
    bih = b_ih.reshape(1, 3 * H)
    bhh = b_hh.reshape(1, 3 * H)
    win_t = W_in.T
    bin_ = b_in.reshape(1, H)
    wmid_t = W_mid.T
    bmid = b_mid.reshape(1, H)
    wout_tp = jnp.zeros((H, H), f32).at[:, :C].set(W_out.T)
    bout_p = jnp.zeros((1, H), f32).at[0, :C].set(b_out)

    m = _mm(h, conv_weight[0])
    for i in range(L):
        parts = _edge_call(m, src3, dst3, ew3, zrows)
        wnext = conv_weight[(i + 1) % L]
        h, m = _gru(parts, h, wih_t, whh_t, bih, bhh, wnext)

    out_p = _pool_head(h, batch3, win_t, bin_, wmid_t, bmid, wout_tp, bout_p)
    return out_p[:, :C]
